# baseline (speedup 1.0000x reference)
import functools

import jax
import jax.numpy as jnp
from jax import lax
from jax.experimental import pallas as pl
from jax.experimental.pallas import tpu as pltpu, tpu_sc as plsc

WIDTH = 1024
TOTAL_ROWS = 4 * 4096

_info = plsc.get_sparse_core_info()
_NC, _NS = _info.num_cores, _info.num_subcores
NUM_WORKERS = _NC * _NS
ROWS_PER_WORKER = TOTAL_ROWS // NUM_WORKERS
CHUNK = 32
NUM_CHUNKS = ROWS_PER_WORKER // CHUNK

_mesh = plsc.VectorSubcoreMesh(core_axis_name="c", subcore_axis_name="s")


@functools.partial(
    pl.kernel,
    mesh=_mesh,
    out_type=jax.ShapeDtypeStruct((TOTAL_ROWS, WIDTH), jnp.float32),
    scratch_types=[
        pltpu.VMEM((NUM_CHUNKS, CHUNK), jnp.int32),
        pltpu.VMEM((CHUNK, WIDTH), jnp.float32),
        pltpu.VMEM((CHUNK, WIDTH), jnp.float32),
        pltpu.SemaphoreType.DMA,
        pltpu.SemaphoreType.DMA,
        pltpu.SemaphoreType.DMA,
        pltpu.SemaphoreType.DMA,
    ],
)
def _lookup_kernel(ids_hbm, table_hbm, out_hbm, idx_v, buf_a, buf_b,
                   gsem_a, gsem_b, ssem_a, ssem_b):
    wid = lax.axis_index("s") * _NC + lax.axis_index("c")
    base = wid * ROWS_PER_WORKER

    pltpu.sync_copy(ids_hbm.at[wid], idx_v)

    bufs = (buf_a, buf_b)
    gsems = (gsem_a, gsem_b)
    ssems = (ssem_a, ssem_b)

    gather = pltpu.async_copy(table_hbm.at[idx_v.at[0]], bufs[0], gsems[0])
    pending_store = [None, None]
    for j in range(NUM_CHUNKS):
        cur = j & 1
        nxt = 1 - cur
        gather.wait()
        if j + 1 < NUM_CHUNKS:
            if pending_store[nxt] is not None:
                pending_store[nxt].wait()
            gather = pltpu.async_copy(
                table_hbm.at[idx_v.at[j + 1]], bufs[nxt], gsems[nxt])
        pending_store[cur] = pltpu.async_copy(
            bufs[cur], out_hbm.at[pl.ds(base + j * CHUNK, CHUNK)], ssems[cur])
    pending_store[0].wait()
    pending_store[1].wait()


def kernel(token_type_ids, token_type_table):
    ids = token_type_ids.reshape(-1).astype(jnp.int32)
    ids = ids.reshape(NUM_WORKERS, NUM_CHUNKS, CHUNK)
    return _lookup_kernel(ids, token_type_table)


# speedup vs baseline: 5.9769x; 5.9769x over previous
import functools

import jax
import jax.numpy as jnp
from jax import lax
from jax.experimental import pallas as pl
from jax.experimental.pallas import tpu as pltpu, tpu_sc as plsc

WIDTH = 1024
TOTAL_ROWS = 4 * 4096

_info = plsc.get_sparse_core_info()
_NC, _NS = _info.num_cores, _info.num_subcores
NUM_WORKERS = _NC * _NS
ROWS_PER_WORKER = TOTAL_ROWS // NUM_WORKERS
CHUNK = 32
NUM_CHUNKS = ROWS_PER_WORKER // CHUNK
REPLICAS = 32
REP_ROWS = 2 * REPLICAS

_mesh = plsc.VectorSubcoreMesh(core_axis_name="c", subcore_axis_name="s")


@functools.partial(
    pl.kernel,
    mesh=_mesh,
    out_type=(
        jax.ShapeDtypeStruct((TOTAL_ROWS, WIDTH), jnp.float32),
        jax.ShapeDtypeStruct((NUM_WORKERS * REP_ROWS, WIDTH), jnp.float32),
    ),
    scratch_types=[
        pltpu.VMEM((2, WIDTH), jnp.float32),
        pltpu.VMEM((NUM_CHUNKS, CHUNK), jnp.int32),
        pltpu.VMEM((CHUNK, WIDTH), jnp.float32),
        pltpu.VMEM((CHUNK, WIDTH), jnp.float32),
        pltpu.SemaphoreType.DMA,
        pltpu.SemaphoreType.DMA,
        pltpu.SemaphoreType.DMA,
        pltpu.SemaphoreType.DMA,
        pltpu.SemaphoreType.DMA,
    ],
)
def _lookup_kernel(ids_hbm, table_hbm, out_hbm, rep_hbm, table_v, idx_v,
                   buf_a, buf_b, rsem, gsem_a, gsem_b, ssem_a, ssem_b):
    wid = lax.axis_index("s") * _NC + lax.axis_index("c")
    base = wid * ROWS_PER_WORKER
    rep_base = wid * REP_ROWS

    pltpu.sync_copy(ids_hbm.at[wid], idx_v)
    pltpu.sync_copy(table_hbm, table_v)

    reps = [
        pltpu.async_copy(table_v, rep_hbm.at[pl.ds(rep_base + 2 * r, 2)], rsem)
        for r in range(REPLICAS)
    ]

    iota = lax.iota(jnp.int32, 16)
    lo = rep_base + 2 * iota
    hi = lo + 32
    for j in range(NUM_CHUNKS):
        idx_v[j, pl.ds(0, 16)] = idx_v[j, pl.ds(0, 16)] + lo
        idx_v[j, pl.ds(16, 16)] = idx_v[j, pl.ds(16, 16)] + hi
    for r in reps:
        r.wait()

    bufs = (buf_a, buf_b)
    gsems = (gsem_a, gsem_b)
    ssems = (ssem_a, ssem_b)

    gather = pltpu.async_copy(rep_hbm.at[idx_v.at[0]], bufs[0], gsems[0])
    pending_store = [None, None]
    for j in range(NUM_CHUNKS):
        cur = j & 1
        nxt = 1 - cur
        gather.wait()
        if j + 1 < NUM_CHUNKS:
            if pending_store[nxt] is not None:
                pending_store[nxt].wait()
            gather = pltpu.async_copy(
                rep_hbm.at[idx_v.at[j + 1]], bufs[nxt], gsems[nxt])
        pending_store[cur] = pltpu.async_copy(
            bufs[cur], out_hbm.at[pl.ds(base + j * CHUNK, CHUNK)], ssems[cur])
    pending_store[0].wait()
    pending_store[1].wait()


def kernel(token_type_ids, token_type_table):
    ids = token_type_ids.reshape(-1).astype(jnp.int32)
    ids = ids.reshape(NUM_WORKERS, NUM_CHUNKS, CHUNK)
    out, _ = _lookup_kernel(ids, token_type_table)
    return out
